# Initial kernel scaffold; baseline (speedup 1.0000x reference)
import jax
import jax.numpy as jnp
from jax import lax
from jax.experimental import pallas as pl
from jax.experimental.pallas import tpu as pltpu
from jax.experimental.pallas import tpu_sc as plsc

B = 4096
N = 50
D = 64
NC = 2
NS = 16
NW = NC * NS

CHUNK = 128

ITEM_ROWS = 2 * N * B
ITEM_PER_W = ITEM_ROWS // NW
ITEM_CHUNKS = ITEM_PER_W // CHUNK
USER_ROWS = N * B + B
USER_PER_W = USER_ROWS // NW
USER_CHUNKS = USER_PER_W // CHUNK


def _sc_gather_body(item_emb, user_emb, item_idx, user_idx,
                    gitem, guser, idx_i_v, idx_u_v, rows_v, sem):
    wid = lax.axis_index("s") * NC + lax.axis_index("c")
    pltpu.sync_copy(item_idx.at[wid], idx_i_v)
    pltpu.sync_copy(user_idx.at[wid], idx_u_v)

    def item_step(c, carry):
        pltpu.async_copy(item_emb.at[idx_i_v.at[c]], rows_v, sem).wait()
        pltpu.sync_copy(rows_v, gitem.at[pl.ds(wid * ITEM_PER_W + c * CHUNK, CHUNK)])
        return carry

    lax.fori_loop(0, ITEM_CHUNKS, item_step, 0)

    def user_step(c, carry):
        pltpu.async_copy(user_emb.at[idx_u_v.at[c]], rows_v, sem).wait()
        pltpu.sync_copy(rows_v, guser.at[pl.ds(wid * USER_PER_W + c * CHUNK, CHUNK)])
        return carry

    lax.fori_loop(0, USER_CHUNKS, user_step, 0)


def _make_sc_gather():
    mesh = plsc.VectorSubcoreMesh(core_axis_name="c", subcore_axis_name="s")
    return pl.kernel(
        _sc_gather_body,
        out_type=(
            jax.ShapeDtypeStruct((ITEM_ROWS, D), jnp.float32),
            jax.ShapeDtypeStruct((USER_ROWS, D), jnp.float32),
        ),
        mesh=mesh,
        scratch_types=[
            pltpu.VMEM((ITEM_CHUNKS, CHUNK), jnp.int32),
            pltpu.VMEM((USER_CHUNKS, CHUNK), jnp.int32),
            pltpu.VMEM((CHUNK, D), jnp.float32),
            pltpu.SemaphoreType.DMA,
        ],
    )


BB = 256


def _tc_gat_body(gitem_ref, guser_ref, aw_ref, av_ref, rw_ref, out_ref):
    aw = aw_ref[...]
    av = av_ref[...]
    rw = rw_ref[...]

    def gat(emb3):
        embf = emb3.reshape(N * BB, D)
        t = jnp.tanh(jnp.dot(embf, aw, preferred_element_type=jnp.float32))
        s3 = jnp.sum(t.reshape(N, BB, D) * av.reshape(1, 1, D), axis=2)
        m = jnp.max(s3, axis=0, keepdims=True)
        e = jnp.exp(s3 - m)
        alpha = e / jnp.sum(e, axis=0, keepdims=True)
        return jnp.sum(emb3 * alpha[:, :, None], axis=0)

    agg1 = gat(gitem_ref[0:N])
    agg3 = gat(gitem_ref[N:2 * N])
    agg2 = gat(guser_ref[0:N])
    tgt = guser_ref[N]

    acc = (jnp.dot(agg1, rw[0:D], preferred_element_type=jnp.float32)
           + jnp.dot(agg2, rw[D:2 * D], preferred_element_type=jnp.float32)
           + jnp.dot(agg3, rw[2 * D:3 * D], preferred_element_type=jnp.float32)
           + jnp.dot(tgt, rw[3 * D:4 * D], preferred_element_type=jnp.float32))
    out_ref[...] = jnp.tanh(acc)


def _tc_gat(gitem_r, guser_r, att_w, av2, refine_w):
    grid = (B // BB,)
    return pl.pallas_call(
        _tc_gat_body,
        grid=grid,
        in_specs=[
            pl.BlockSpec((2 * N, BB, D), lambda i: (0, i, 0)),
            pl.BlockSpec((N + 1, BB, D), lambda i: (0, i, 0)),
            pl.BlockSpec((D, D), lambda i: (0, 0)),
            pl.BlockSpec((1, D), lambda i: (0, 0)),
            pl.BlockSpec((4 * D, D), lambda i: (0, 0)),
        ],
        out_specs=pl.BlockSpec((BB, D), lambda i: (i, 0)),
        out_shape=jax.ShapeDtypeStruct((B, D), jnp.float32),
    )(gitem_r, guser_r, att_w, av2, refine_w)


def kernel(target_ids, support_1st, support_2nd, support_3rd,
           user_emb, item_emb, att_w, att_v, refine_w):
    item_idx = jnp.concatenate(
        [support_1st.T.reshape(-1), support_3rd.T.reshape(-1)]
    ).reshape(NW, ITEM_CHUNKS, CHUNK)
    user_idx = jnp.concatenate(
        [support_2nd.T.reshape(-1), target_ids]
    ).reshape(NW, USER_CHUNKS, CHUNK)

    gitem, guser = _make_sc_gather()(item_emb, user_emb, item_idx, user_idx)

    gitem_r = gitem.reshape(2 * N, B, D)
    guser_r = guser.reshape(N + 1, B, D)
    return _tc_gat(gitem_r, guser_r, att_w, att_v.reshape(1, D), refine_w)


# speedup vs baseline: 1.2409x; 1.2409x over previous
import jax
import jax.numpy as jnp
from jax import lax
from jax.experimental import pallas as pl
from jax.experimental.pallas import tpu as pltpu
from jax.experimental.pallas import tpu_sc as plsc

B = 4096
N = 50
D = 64
NC = 2
NS = 16
NW = NC * NS

CHUNK = 128

ITEM_ROWS = 2 * N * B
ITEM_PER_W = ITEM_ROWS // NW
ITEM_CHUNKS = ITEM_PER_W // CHUNK
USER_ROWS = N * B + B
USER_PER_W = USER_ROWS // NW
USER_CHUNKS = USER_PER_W // CHUNK


def _sc_gather_body(item_emb, user_emb, item_idx, user_idx,
                    gitem, guser, idx_i_v, idx_u_v, rows_v, sem):
    wid = lax.axis_index("s") * NC + lax.axis_index("c")
    pltpu.sync_copy(item_idx.at[wid], idx_i_v)
    pltpu.sync_copy(user_idx.at[wid], idx_u_v)

    def item_step(c, carry):
        pltpu.async_copy(item_emb.at[idx_i_v.at[c]], rows_v, sem).wait()
        pltpu.sync_copy(rows_v, gitem.at[pl.ds(wid * ITEM_PER_W + c * CHUNK, CHUNK)])
        return carry

    lax.fori_loop(0, ITEM_CHUNKS, item_step, 0)

    def user_step(c, carry):
        pltpu.async_copy(user_emb.at[idx_u_v.at[c]], rows_v, sem).wait()
        pltpu.sync_copy(rows_v, guser.at[pl.ds(wid * USER_PER_W + c * CHUNK, CHUNK)])
        return carry

    lax.fori_loop(0, USER_CHUNKS, user_step, 0)


def _make_sc_gather():
    mesh = plsc.VectorSubcoreMesh(core_axis_name="c", subcore_axis_name="s")
    return pl.kernel(
        _sc_gather_body,
        out_type=(
            jax.ShapeDtypeStruct((ITEM_ROWS, D), jnp.float32),
            jax.ShapeDtypeStruct((USER_ROWS, D), jnp.float32),
        ),
        mesh=mesh,
        scratch_types=[
            pltpu.VMEM((ITEM_CHUNKS, CHUNK), jnp.int32),
            pltpu.VMEM((USER_CHUNKS, CHUNK), jnp.int32),
            pltpu.VMEM((CHUNK, D), jnp.float32),
            pltpu.SemaphoreType.DMA,
        ],
        compiler_params=pltpu.CompilerParams(use_tc_tiling_on_sc=False),
    )


BB = 256


def _tc_gat_body(gitem_ref, guser_ref, aw_ref, av_ref, rw_ref, out_ref):
    aw = aw_ref[...]
    av = av_ref[...]
    rw = rw_ref[...]

    def gat(emb3):
        embf = emb3.reshape(N * BB, D)
        t = jnp.tanh(jnp.dot(embf, aw, preferred_element_type=jnp.float32))
        s3 = jnp.sum(t.reshape(N, BB, D) * av.reshape(1, 1, D), axis=2)
        m = jnp.max(s3, axis=0, keepdims=True)
        e = jnp.exp(s3 - m)
        alpha = e / jnp.sum(e, axis=0, keepdims=True)
        return jnp.sum(emb3 * alpha[:, :, None], axis=0)

    agg1 = gat(gitem_ref[0:N])
    agg3 = gat(gitem_ref[N:2 * N])
    agg2 = gat(guser_ref[0:N])
    tgt = guser_ref[N]

    acc = (jnp.dot(agg1, rw[0:D], preferred_element_type=jnp.float32)
           + jnp.dot(agg2, rw[D:2 * D], preferred_element_type=jnp.float32)
           + jnp.dot(agg3, rw[2 * D:3 * D], preferred_element_type=jnp.float32)
           + jnp.dot(tgt, rw[3 * D:4 * D], preferred_element_type=jnp.float32))
    out_ref[...] = jnp.tanh(acc)


def _tc_gat(gitem_r, guser_r, att_w, av2, refine_w):
    grid = (B // BB,)
    return pl.pallas_call(
        _tc_gat_body,
        grid=grid,
        in_specs=[
            pl.BlockSpec((2 * N, BB, D), lambda i: (0, i, 0)),
            pl.BlockSpec((N + 1, BB, D), lambda i: (0, i, 0)),
            pl.BlockSpec((D, D), lambda i: (0, 0)),
            pl.BlockSpec((1, D), lambda i: (0, 0)),
            pl.BlockSpec((4 * D, D), lambda i: (0, 0)),
        ],
        out_specs=pl.BlockSpec((BB, D), lambda i: (i, 0)),
        out_shape=jax.ShapeDtypeStruct((B, D), jnp.float32),
    )(gitem_r, guser_r, att_w, av2, refine_w)


def kernel(target_ids, support_1st, support_2nd, support_3rd,
           user_emb, item_emb, att_w, att_v, refine_w):
    item_idx = jnp.concatenate(
        [support_1st.T.reshape(-1), support_3rd.T.reshape(-1)]
    ).reshape(NW, ITEM_CHUNKS, CHUNK)
    user_idx = jnp.concatenate(
        [support_2nd.T.reshape(-1), target_ids]
    ).reshape(NW, USER_CHUNKS, CHUNK)

    gitem, guser = _make_sc_gather()(item_emb, user_emb, item_idx, user_idx)

    gitem_r = gitem.reshape(2 * N, B, D)
    guser_r = guser.reshape(N + 1, B, D)
    return _tc_gat(gitem_r, guser_r, att_w, att_v.reshape(1, D), refine_w)
